# baseline (speedup 1.0000x reference)
import functools

import jax
import jax.numpy as jnp
from jax import lax
from jax.experimental import pallas as pl
from jax.experimental.pallas import tpu as pltpu
from jax.experimental.pallas import tpu_sc as plsc

VOCAB = 1000000
D = 64
B = 4096
H = 50
LANES = 16
D_VREGS = D // LANES

NC = 2
NS = 16
NW = NC * NS
ROWS_PER_W = B // NW
GROUP = 2
IDX_PER_G = GROUP * H
G_PER_W = ROWS_PER_W // GROUP


def _sc_pool_sums(x2, table):
  mesh = plsc.VectorSubcoreMesh(
      core_axis_name="c", subcore_axis_name="s", num_cores=NC, num_subcores=NS
  )

  @functools.partial(
      pl.kernel,
      out_type=jax.ShapeDtypeStruct((B, D), jnp.float32),
      mesh=mesh,
      compiler_params=pltpu.CompilerParams(use_tc_tiling_on_sc=False),
      scratch_types=[
          pltpu.VMEM((G_PER_W, IDX_PER_G), jnp.int32),
          pltpu.VMEM((IDX_PER_G, D), jnp.float32),
          pltpu.VMEM((ROWS_PER_W, D), jnp.float32),
          pltpu.SemaphoreType.DMA,
      ],
  )
  def k(x_hbm, tab_hbm, out_hbm, idx_v, rows_v, out_v, sem):
    wid = lax.axis_index("s") * NC + lax.axis_index("c")
    gbase = wid * G_PER_W

    pltpu.sync_copy(x_hbm.at[pl.ds(gbase, G_PER_W)], idx_v)

    def group_body(g, carry):
      pltpu.async_copy(tab_hbm.at[idx_v.at[g]], rows_v, sem).wait()
      for j in range(GROUP):
        def add_row(r, accs):
          return tuple(
              accs[c] + rows_v[j * H + r, pl.ds(c * LANES, LANES)]
              for c in range(D_VREGS)
          )
        accs = lax.fori_loop(
            0, H, add_row,
            tuple(jnp.zeros((LANES,), jnp.float32) for _ in range(D_VREGS)),
        )
        for c in range(D_VREGS):
          out_v[g * GROUP + j, pl.ds(c * LANES, LANES)] = accs[c]
      return carry

    lax.fori_loop(0, G_PER_W, group_body, 0)
    pltpu.sync_copy(out_v, out_hbm.at[pl.ds(wid * ROWS_PER_W, ROWS_PER_W)])

  return k(x2, table)


def _normalize(sums):
  def body(s_ref, o_ref):
    p = s_ref[...] * (1.0 / H)
    ss = jnp.sum(p * p, axis=1, keepdims=True)
    denom = jnp.maximum(jnp.sqrt(ss), 1e-12)
    o_ref[...] = p / denom

  return pl.pallas_call(
      body,
      out_shape=jax.ShapeDtypeStruct((B, D), jnp.float32),
  )(sums)


@jax.jit
def kernel(x, table):
  x2 = x.astype(jnp.int32).reshape(B // GROUP, IDX_PER_G)
  sums = _sc_pool_sums(x2, table)
  return _normalize(sums)


# speedup vs baseline: 1.0899x; 1.0899x over previous
import functools

import jax
import jax.numpy as jnp
from jax import lax
from jax.experimental import pallas as pl
from jax.experimental.pallas import tpu as pltpu
from jax.experimental.pallas import tpu_sc as plsc

VOCAB = 1000000
D = 64
B = 4096
H = 50
LANES = 16
D_VREGS = D // LANES

NC = 2
NS = 16
NW = NC * NS
ROWS_PER_W = B // NW
GROUP = 2
IDX_PER_G = GROUP * H
G_PER_W = ROWS_PER_W // GROUP
NBUF = 8


def _sc_pool_sums(x2, table):
  mesh = plsc.VectorSubcoreMesh(
      core_axis_name="c", subcore_axis_name="s", num_cores=NC, num_subcores=NS
  )

  @functools.partial(
      pl.kernel,
      out_type=jax.ShapeDtypeStruct((B, D), jnp.float32),
      mesh=mesh,
      compiler_params=pltpu.CompilerParams(use_tc_tiling_on_sc=False),
      scratch_types=[
          pltpu.VMEM((G_PER_W, IDX_PER_G), jnp.int32),
          pltpu.VMEM((NBUF, IDX_PER_G, D), jnp.float32),
          pltpu.VMEM((ROWS_PER_W, D), jnp.float32),
          [pltpu.SemaphoreType.DMA] * NBUF,
      ],
  )
  def k(x_hbm, tab_hbm, out_hbm, idx_v, rows_v, out_v, sems):
    wid = lax.axis_index("s") * NC + lax.axis_index("c")
    gbase = wid * G_PER_W

    pltpu.sync_copy(x_hbm.at[pl.ds(gbase, G_PER_W)], idx_v)

    for b in range(NBUF):
      pltpu.async_copy(tab_hbm.at[idx_v.at[b]], rows_v.at[b], sems[b])

    def accumulate(b, g):
      for j in range(GROUP):
        def add_rows(r, accs):
          return tuple(
              accs[c]
              + rows_v[b, j * H + 2 * r, pl.ds(c * LANES, LANES)]
              + rows_v[b, j * H + 2 * r + 1, pl.ds(c * LANES, LANES)]
              for c in range(D_VREGS)
          )
        accs = lax.fori_loop(
            0, H // 2, add_rows,
            tuple(jnp.zeros((LANES,), jnp.float32) for _ in range(D_VREGS)),
        )
        for c in range(D_VREGS):
          out_v[g * GROUP + j, pl.ds(c * LANES, LANES)] = accs[c]

    def outer(t, carry):
      for b in range(NBUF):
        g = t * NBUF + b
        pltpu.make_async_copy(
            tab_hbm.at[idx_v.at[g]], rows_v.at[b], sems[b]
        ).wait()
        accumulate(b, g)
        nxt = g + NBUF

        @pl.when(nxt < G_PER_W)
        def _():
          pltpu.async_copy(tab_hbm.at[idx_v.at[nxt]], rows_v.at[b], sems[b])

      return carry

    lax.fori_loop(0, G_PER_W // NBUF, outer, 0)
    pltpu.sync_copy(out_v, out_hbm.at[pl.ds(wid * ROWS_PER_W, ROWS_PER_W)])

  return k(x2, table)


def _normalize(sums):
  def body(s_ref, o_ref):
    p = s_ref[...] * (1.0 / H)
    ss = jnp.sum(p * p, axis=1, keepdims=True)
    denom = jnp.maximum(jnp.sqrt(ss), 1e-12)
    o_ref[...] = p / denom

  return pl.pallas_call(
      body,
      out_shape=jax.ShapeDtypeStruct((B, D), jnp.float32),
  )(sums)


@jax.jit
def kernel(x, table):
  x2 = x.astype(jnp.int32).reshape(B // GROUP, IDX_PER_G)
  sums = _sc_pool_sums(x2, table)
  return _normalize(sums)
